# baseline (speedup 1.0000x reference)
import functools

import jax
import jax.numpy as jnp
from jax import lax
from jax.experimental import pallas as pl
from jax.experimental.pallas import tpu as pltpu
from jax.experimental.pallas import tpu_sc as plsc

_B = 16384
_D = 64
_V_ITEM = 1000000
_V_BRAND = 100000
_V_CAT = 1000
_TEXT_DIM = 768


def _gather_body(n_tables, nc, bpw, args):
    idx_hbms = args[:n_tables]
    tabs = args[n_tables:2 * n_tables]
    outs = args[2 * n_tables:3 * n_tables]
    iv, rows, sem = args[3 * n_tables:]
    wid = lax.axis_index("s") * nc + lax.axis_index("c")
    base = wid * bpw
    for idx_hbm, tab, out in zip(idx_hbms, tabs, outs):
        pltpu.sync_copy(idx_hbm.at[pl.ds(base, bpw)], iv)

        def body(k, _):
            vec = iv[pl.ds(k * 16, 16)]
            for j in range(16):
                pltpu.async_copy(tab.at[pl.ds(vec[j], 1)],
                                 rows.at[pl.ds(k * 16 + j, 1)], sem)
            return _

        lax.fori_loop(0, bpw // 16, body, 0)
        pltpu.make_async_copy(tab.at[pl.ds(0, bpw)], rows, sem).wait()
        pltpu.sync_copy(rows, out.at[pl.ds(base, bpw)])


@functools.cache
def _make_gather(n_tables):
    info = plsc.get_sparse_core_info()
    nc, ns = info.num_cores, info.num_subcores
    bpw = _B // (nc * ns)

    mesh = plsc.VectorSubcoreMesh(core_axis_name="c", subcore_axis_name="s")

    @functools.partial(
        pl.kernel,
        mesh=mesh,
        out_type=[jax.ShapeDtypeStruct((_B, _D), jnp.float32)] * n_tables,
        scratch_types=[
            pltpu.VMEM((bpw,), jnp.int32),
            pltpu.VMEM((bpw, _D), jnp.float32),
            pltpu.SemaphoreType.DMA,
        ],
    )
    def gather(*args):
        _gather_body(n_tables, nc, bpw, args)

    return gather


def _mlp_body(text_ref, ei_ref, eb_ref, ec_ref,
              wt_ref, bt_ref, w1_ref, b1_ref, w2_ref, b2_ref, w3_ref, b3_ref,
              out_ref):
    e_text = (jnp.dot(text_ref[...], wt_ref[...],
                      preferred_element_type=jnp.float32) + bt_ref[...])
    x = jnp.concatenate([ei_ref[...], eb_ref[...], ec_ref[...], e_text],
                        axis=-1)
    h = jnp.maximum(jnp.dot(x, w1_ref[...],
                            preferred_element_type=jnp.float32) + b1_ref[...],
                    0.0)
    h = jnp.maximum(jnp.dot(h, w2_ref[...],
                            preferred_element_type=jnp.float32) + b2_ref[...],
                    0.0)
    o = (jnp.dot(h, w3_ref[...], preferred_element_type=jnp.float32)
         + b3_ref[...])
    n = jnp.maximum(jnp.sqrt(jnp.sum(o * o, axis=1, keepdims=True)), 1e-12)
    out_ref[...] = o / n


def kernel(item_id, brand, category, text_features, emb_item_id, emb_brand,
           emb_category, W_text, b_text, W1, b1, W2, b2, W3, b3):
    ii = jnp.clip(item_id, 0, _V_ITEM - 1)
    bb = jnp.clip(brand, 0, _V_BRAND - 1)
    cc = jnp.clip(category, 0, _V_CAT - 1)

    e_brand, e_cat = _make_gather(2)(bb, cc, emb_brand, emb_category)
    (e_item,) = _make_gather(1)(ii, emb_item_id)

    blk = 4096
    grid = (_B // blk,)

    def b_spec(w):
        return pl.BlockSpec((blk, w), lambda i: (i, 0))

    def w_spec(shape):
        return pl.BlockSpec(shape, lambda i: (0, 0))

    out = pl.pallas_call(
        _mlp_body,
        grid=grid,
        in_specs=[
            b_spec(_TEXT_DIM),
            b_spec(_D), b_spec(_D), b_spec(_D),
            w_spec((_TEXT_DIM, _D)), w_spec((1, _D)),
            w_spec((4 * _D, 256)), w_spec((1, 256)),
            w_spec((256, 128)), w_spec((1, 128)),
            w_spec((128, _D)), w_spec((1, _D)),
        ],
        out_specs=b_spec(_D),
        out_shape=jax.ShapeDtypeStruct((_B, _D), jnp.float32),
    )(text_features, e_item, e_brand, e_cat,
      W_text, b_text.reshape(1, _D), W1, b1.reshape(1, 256),
      W2, b2.reshape(1, 128), W3, b3.reshape(1, _D))
    return out
